# baseline (speedup 1.0000x reference)
import functools

import jax
import jax.numpy as jnp
from jax import lax
from jax.experimental import pallas as pl
from jax.experimental.pallas import tpu as pltpu
from jax.experimental.pallas import tpu_sc as plsc

N = 10000
D = 128
G = 64
E = 320000
EN = E + N
L = 5
NUM_ATOM_TYPE = 119
NUM_CHIRALITY = 3

NC = 2
NS = 16
NW = NC * NS
CHUNK = 128
CPW = 82
EP = NW * CPW * CHUNK
EPW = CPW * CHUNK
NP = 10240
RPS = NP // NS
RC = RPS // CHUNK


def _sc_edge_body(xw_hbm, row_hbm, col_hbm, zeros_hbm, out0_hbm, out1_hbm,
                  acc_sh, row_v, colc_v, gat0_v, gat1_v,
                  sg0, sg1, ss0, ss1, si0, si1):
    c = lax.axis_index("c")
    s = lax.axis_index("s")
    w = c * NS + s
    gat = (gat0_v, gat1_v)
    sg = (sg0, sg1)
    ss = (ss0, ss1)
    si = (si0, si1)
    pltpu.sync_copy(zeros_hbm, gat0_v)
    for j in range(RC):
        pltpu.sync_copy(gat0_v, acc_sh.at[pl.ds(s * RPS + j * CHUNK, CHUNK)])
    plsc.subcore_barrier()
    pltpu.sync_copy(row_hbm.at[w], row_v)

    for b in range(2):
        pltpu.async_copy(col_hbm.at[w, b], colc_v.at[b], si[b])
        pltpu.async_copy(xw_hbm.at[row_v.at[b]], gat[b], sg[b])

    def body(g, carry):
        for b in range(2):
            j = 2 * g + b
            pltpu.make_async_copy(col_hbm.at[w, j], colc_v.at[b], si[b]).wait()
            pltpu.make_async_copy(xw_hbm.at[row_v.at[j]], gat[b], sg[b]).wait()
            pltpu.async_copy(gat[b], acc_sh.at[colc_v.at[b]], ss[b], add=True)
            pltpu.make_async_copy(gat[b], acc_sh.at[colc_v.at[b]], ss[b]).wait()
            pltpu.async_copy(col_hbm.at[w, j + 2], colc_v.at[b], si[b])
            pltpu.async_copy(xw_hbm.at[row_v.at[j + 2]], gat[b], sg[b])
        return carry

    lax.fori_loop(0, CPW // 2 - 1, body, 0)
    for b in range(2):
        j = CPW - 2 + b
        pltpu.make_async_copy(col_hbm.at[w, j], colc_v.at[b], si[b]).wait()
        pltpu.make_async_copy(xw_hbm.at[row_v.at[j]], gat[b], sg[b]).wait()
        pltpu.sync_copy(gat[b], acc_sh.at[colc_v.at[b]], add=True)
    plsc.subcore_barrier()

    @pl.when(c == 0)
    def _():
        for j in range(RC):
            pltpu.sync_copy(acc_sh.at[pl.ds(s * RPS + j * CHUNK, CHUNK)], gat0_v)
            pltpu.sync_copy(gat0_v, out0_hbm.at[pl.ds(s * RPS + j * CHUNK, CHUNK)])

    @pl.when(c == 1)
    def _():
        for j in range(RC):
            pltpu.sync_copy(acc_sh.at[pl.ds(s * RPS + j * CHUNK, CHUNK)], gat0_v)
            pltpu.sync_copy(gat0_v, out1_hbm.at[pl.ds(s * RPS + j * CHUNK, CHUNK)])


@functools.cache
def _get_sc_edge():
    return functools.partial(
        pl.kernel,
        out_type=[jax.ShapeDtypeStruct((NP, D), jnp.float32),
                  jax.ShapeDtypeStruct((NP, D), jnp.float32)],
        mesh=plsc.VectorSubcoreMesh(core_axis_name="c", subcore_axis_name="s",
                                    num_cores=NC, num_subcores=NS),
        scratch_types=[
            pltpu.VMEM_SHARED((NP, D), jnp.float32),
            pltpu.VMEM((CPW, CHUNK), jnp.int32),
            pltpu.VMEM((2, CHUNK), jnp.int32),
            pltpu.VMEM((CHUNK, D), jnp.float32),
            pltpu.VMEM((CHUNK, D), jnp.float32),
            pltpu.SemaphoreType.DMA,
            pltpu.SemaphoreType.DMA,
            pltpu.SemaphoreType.DMA,
            pltpu.SemaphoreType.DMA,
            pltpu.SemaphoreType.DMA,
            pltpu.SemaphoreType.DMA,
        ],
    )(_sc_edge_body)


def _tc_embed_body(x0_ref, x1_ref, emb1_ref, emb2_ref, w_ref, out_ref):
    x0 = x0_ref[...]
    x1 = x1_ref[...]
    i1 = lax.broadcasted_iota(jnp.int32, (NP, NUM_ATOM_TYPE), 1)
    oh1 = (x0 == i1).astype(jnp.float32)
    i2 = lax.broadcasted_iota(jnp.int32, (NP, NUM_CHIRALITY), 1)
    oh2 = (x1 == i2).astype(jnp.float32)
    h0 = (jnp.dot(oh1, emb1_ref[...], preferred_element_type=jnp.float32, precision=lax.Precision.HIGHEST)
          + jnp.dot(oh2, emb2_ref[...], preferred_element_type=jnp.float32, precision=lax.Precision.HIGHEST))
    out_ref[...] = jnp.dot(h0, w_ref[...], preferred_element_type=jnp.float32)


def _tc_ee_body(ea0_ref, ea1_ref, t1_ref, t2_ref, out_ref):
    ea0 = ea0_ref[...]
    ea1 = ea1_ref[...]
    acc = jnp.zeros(out_ref.shape, jnp.float32)
    for k in range(5):
        acc += jnp.where(ea0 == k, 1.0, 0.0) * t1_ref[k, :][None, :]
    for k in range(3):
        acc += jnp.where(ea1 == k, 1.0, 0.0) * t2_ref[k, :][None, :]
    out_ref[...] = acc


def _tc_bn_body(l, p0_ref, p1_ref, es0_ref, es1_ref, b_ref, g_ref, be_ref,
                w_ref, out_ref):
    agg = (p0_ref[...] + p1_ref[...] + b_ref[...]
           + (es0_ref[:, l:l + 1] + es1_ref[:, l:l + 1]))
    live = agg[:N]
    mean = jnp.mean(live, axis=0, keepdims=True)
    var = jnp.mean((live - mean) ** 2, axis=0, keepdims=True)
    h = (agg - mean) * lax.rsqrt(var + 1e-5) * g_ref[...] + be_ref[...]
    if l < L - 1:
        h = jnp.maximum(h, 0.0)
        out_ref[...] = jnp.dot(h, w_ref[...], preferred_element_type=jnp.float32)
    else:
        out_ref[...] = h


def _tc_pool_body(h_ref, batch_ref, fw_ref, fb_ref, w1_ref, b1_ref,
                  w2_ref, b2_ref, out_ref):
    bt = batch_ref[...]
    gi = lax.broadcasted_iota(jnp.int32, (G, NP), 0)
    oh = (gi == bt).astype(jnp.float32)
    sums = jnp.dot(oh, h_ref[...], preferred_element_type=jnp.float32, precision=lax.Precision.HIGHEST)
    counts = jnp.sum(oh, axis=1, keepdims=True)
    pooled = sums / jnp.maximum(counts, 1.0)
    feat = jnp.dot(pooled, fw_ref[...],
                   preferred_element_type=jnp.float32) + fb_ref[...]
    z = jnp.dot(feat, w1_ref[...],
                preferred_element_type=jnp.float32) + b1_ref[...]
    hdn = jnp.maximum(z, 0.0) + jnp.log1p(jnp.exp(-jnp.abs(z)))
    out_ref[...] = jnp.dot(hdn, w2_ref[...],
                           preferred_element_type=jnp.float32) + b2_ref[...]


def kernel(x, edge_index, edge_attr, batch, params):
    f32 = jnp.float32
    i32 = jnp.int32

    x0p = jnp.pad(x[:, 0:1].astype(i32), ((0, NP - N), (0, 0)))
    x1p = jnp.pad(x[:, 1:2].astype(i32), ((0, NP - N), (0, 0)))

    loop = jnp.arange(N, dtype=i32)
    npad = EP - EN
    pad_row = jnp.arange(npad, dtype=i32) % 16
    pad_col = N + (jnp.arange(npad, dtype=i32) % (NP - N))
    row = jnp.concatenate([edge_index[0].astype(i32), loop, pad_row])
    col = jnp.concatenate([edge_index[1].astype(i32), loop, pad_col])
    row3 = row.reshape(NW, CPW, CHUNK)
    col3 = col.reshape(NW, CPW, CHUNK)

    ea0 = jnp.concatenate([edge_attr[:, 0].astype(i32),
                           jnp.full((N,), 4, i32),
                           jnp.zeros((npad,), i32)])[:, None]
    ea1 = jnp.concatenate([edge_attr[:, 1].astype(i32),
                           jnp.zeros((N,), i32),
                           jnp.zeros((npad,), i32)])[:, None]

    t1 = jnp.concatenate([p['ee1'] for p in params['layers']], axis=1)
    t2 = jnp.concatenate([p['ee2'] for p in params['layers']], axis=1)
    t1 = jnp.pad(t1, ((0, 0), (0, D - L))).astype(f32)
    t2 = jnp.pad(t2, ((0, 0), (0, D - L))).astype(f32)

    zeros128 = jnp.zeros((CHUNK, D), f32)
    batchp = jnp.pad(batch.astype(i32), (0, NP - N),
                     constant_values=G)[None, :]

    ee_all = pl.pallas_call(
        _tc_ee_body,
        grid=(EP // 4096,),
        in_specs=[
            pl.BlockSpec((4096, 1), lambda i: (i, 0)),
            pl.BlockSpec((4096, 1), lambda i: (i, 0)),
            pl.BlockSpec((5, D), lambda i: (0, 0)),
            pl.BlockSpec((3, D), lambda i: (0, 0)),
        ],
        out_specs=pl.BlockSpec((4096, D), lambda i: (i, 0)),
        out_shape=jax.ShapeDtypeStruct((EP, D), f32),
    )(ea0, ea1, t1, t2)

    e3 = jnp.arange(EP, dtype=i32).reshape(NW, CPW, CHUNK)
    es0, es1 = _get_sc_edge()(ee_all, e3, col3, zeros128)

    xw = pl.pallas_call(
        _tc_embed_body,
        out_shape=jax.ShapeDtypeStruct((NP, D), f32),
    )(x0p, x1p, params['emb1'].astype(f32), params['emb2'].astype(f32),
      params['layers'][0]['W'].astype(f32))

    for l in range(L):
        part0, part1 = _get_sc_edge()(xw, row3, col3, zeros128)
        p = params['layers'][l]
        w_next = (params['layers'][l + 1]['W'] if l < L - 1
                  else params['layers'][0]['W']).astype(f32)
        xw = pl.pallas_call(
            functools.partial(_tc_bn_body, l),
            out_shape=jax.ShapeDtypeStruct((NP, D), f32),
        )(part0, part1, es0, es1, p['b'][None, :].astype(f32),
          p['gamma'][None, :].astype(f32), p['beta'][None, :].astype(f32),
          w_next)

    out = pl.pallas_call(
        _tc_pool_body,
        out_shape=jax.ShapeDtypeStruct((G, 1), f32),
    )(xw, batchp, params['feat_W'].astype(f32), params['feat_b'][None, :].astype(f32),
      params['h1_W'].astype(f32), params['h1_b'][None, :].astype(f32),
      params['h2_W'].astype(f32), params['h2_b'][None, :].astype(f32))
    return out


# speedup vs baseline: 1.4362x; 1.4362x over previous
import functools

import jax
import jax.numpy as jnp
from jax import lax
from jax.experimental import pallas as pl
from jax.experimental.pallas import tpu as pltpu
from jax.experimental.pallas import tpu_sc as plsc

N = 10000
D = 128
G = 64
E = 320000
EN = E + N
L = 5
NUM_ATOM_TYPE = 119
NUM_CHIRALITY = 3

NC = 2
NS = 16
NW = NC * NS
CHUNK = 128
CPW = 82
EP = NW * CPW * CHUNK
EPW = CPW * CHUNK
NP = 10240
RPS = NP // NS
RC = RPS // CHUNK


def _sc_edge_body(xw_hbm, row_hbm, col_hbm, zeros_hbm, out_hbm,
                  acc_sh, row_v, colc_v, gat0_v, gat1_v,
                  sg0, sg1, ss0, ss1, si0, si1):
    c = lax.axis_index("c")
    s = lax.axis_index("s")
    w = c * NS + s
    gat = (gat0_v, gat1_v)
    sg = (sg0, sg1)
    ss = (ss0, ss1)
    si = (si0, si1)
    pltpu.sync_copy(zeros_hbm, gat0_v)
    for j in range(RC):
        pltpu.sync_copy(gat0_v, acc_sh.at[pl.ds(s * RPS + j * CHUNK, CHUNK)])
    plsc.subcore_barrier()
    pltpu.sync_copy(row_hbm.at[w], row_v)

    for b in range(2):
        pltpu.async_copy(col_hbm.at[w, b], colc_v.at[b], si[b])
        pltpu.async_copy(xw_hbm.at[row_v.at[b]], gat[b], sg[b])

    def body(g, carry):
        for b in range(2):
            j = 2 * g + b
            pltpu.make_async_copy(col_hbm.at[w, j], colc_v.at[b], si[b]).wait()
            pltpu.make_async_copy(xw_hbm.at[row_v.at[j]], gat[b], sg[b]).wait()
            pltpu.async_copy(gat[b], acc_sh.at[colc_v.at[b]], ss[b], add=True)
            pltpu.make_async_copy(gat[b], acc_sh.at[colc_v.at[b]], ss[b]).wait()
            pltpu.async_copy(col_hbm.at[w, j + 2], colc_v.at[b], si[b])
            pltpu.async_copy(xw_hbm.at[row_v.at[j + 2]], gat[b], sg[b])
        return carry

    lax.fori_loop(0, CPW // 2 - 1, body, 0)
    for b in range(2):
        j = CPW - 2 + b
        pltpu.make_async_copy(col_hbm.at[w, j], colc_v.at[b], si[b]).wait()
        pltpu.make_async_copy(xw_hbm.at[row_v.at[j]], gat[b], sg[b]).wait()
        pltpu.sync_copy(gat[b], acc_sh.at[colc_v.at[b]], add=True)
    plsc.subcore_barrier()
    for j in range(RC):
        pltpu.sync_copy(acc_sh.at[pl.ds(s * RPS + j * CHUNK, CHUNK)], gat0_v)
        pltpu.sync_copy(gat0_v, out_hbm.at[c, pl.ds(s * RPS + j * CHUNK, CHUNK)])


@functools.cache
def _get_sc_edge():
    return functools.partial(
        pl.kernel,
        out_type=jax.ShapeDtypeStruct((NC, NP, D), jnp.float32),
        mesh=plsc.VectorSubcoreMesh(core_axis_name="c", subcore_axis_name="s",
                                    num_cores=NC, num_subcores=NS),
        scratch_types=[
            pltpu.VMEM_SHARED((NP, D), jnp.float32),
            pltpu.VMEM((CPW, CHUNK), jnp.int32),
            pltpu.VMEM((2, CHUNK), jnp.int32),
            pltpu.VMEM((CHUNK, D), jnp.float32),
            pltpu.VMEM((CHUNK, D), jnp.float32),
            pltpu.SemaphoreType.DMA,
            pltpu.SemaphoreType.DMA,
            pltpu.SemaphoreType.DMA,
            pltpu.SemaphoreType.DMA,
            pltpu.SemaphoreType.DMA,
            pltpu.SemaphoreType.DMA,
        ],
    )(_sc_edge_body)


def _tc_embed_body(x0_ref, x1_ref, emb1_ref, emb2_ref, w_ref, out_ref):
    x0 = x0_ref[...]
    x1 = x1_ref[...]
    i1 = lax.broadcasted_iota(jnp.int32, (NP, NUM_ATOM_TYPE), 1)
    oh1 = (x0 == i1).astype(jnp.float32)
    i2 = lax.broadcasted_iota(jnp.int32, (NP, NUM_CHIRALITY), 1)
    oh2 = (x1 == i2).astype(jnp.float32)
    h0 = (jnp.dot(oh1, emb1_ref[...], preferred_element_type=jnp.float32, precision=lax.Precision.HIGHEST)
          + jnp.dot(oh2, emb2_ref[...], preferred_element_type=jnp.float32, precision=lax.Precision.HIGHEST))
    out_ref[...] = jnp.dot(h0, w_ref[...], preferred_element_type=jnp.float32)


def _tc_bn_body(l, p_ref, es_ref, b_ref, g_ref, be_ref, w_ref, out_ref):
    agg = (p_ref[0] + p_ref[1] + b_ref[...]
           + (es_ref[0, :, l:l + 1] + es_ref[1, :, l:l + 1]))
    live = agg[:N]
    mean = jnp.mean(live, axis=0, keepdims=True)
    var = jnp.mean((live - mean) ** 2, axis=0, keepdims=True)
    h = (agg - mean) * lax.rsqrt(var + 1e-5) * g_ref[...] + be_ref[...]
    if l < L - 1:
        h = jnp.maximum(h, 0.0)
        out_ref[...] = jnp.dot(h, w_ref[...], preferred_element_type=jnp.float32)
    else:
        out_ref[...] = h


def _tc_pool_body(h_ref, batch_ref, fw_ref, fb_ref, w1_ref, b1_ref,
                  w2_ref, b2_ref, out_ref):
    bt = batch_ref[...]
    gi = lax.broadcasted_iota(jnp.int32, (G, NP), 0)
    oh = (gi == bt).astype(jnp.float32)
    sums = jnp.dot(oh, h_ref[...], preferred_element_type=jnp.float32, precision=lax.Precision.HIGHEST)
    counts = jnp.sum(oh, axis=1, keepdims=True)
    pooled = sums / jnp.maximum(counts, 1.0)
    feat = jnp.dot(pooled, fw_ref[...],
                   preferred_element_type=jnp.float32) + fb_ref[...]
    z = jnp.dot(feat, w1_ref[...],
                preferred_element_type=jnp.float32) + b1_ref[...]
    hdn = jnp.maximum(z, 0.0) + jnp.log1p(jnp.exp(-jnp.abs(z)))
    out_ref[...] = jnp.dot(hdn, w2_ref[...],
                           preferred_element_type=jnp.float32) + b2_ref[...]


def kernel(x, edge_index, edge_attr, batch, params):
    f32 = jnp.float32
    i32 = jnp.int32

    x0p = jnp.pad(x[:, 0:1].astype(i32), ((0, NP - N), (0, 0)))
    x1p = jnp.pad(x[:, 1:2].astype(i32), ((0, NP - N), (0, 0)))

    loop = jnp.arange(N, dtype=i32)
    npad = EP - EN
    pad_row = jnp.arange(npad, dtype=i32) % 16
    pad_col = N + (jnp.arange(npad, dtype=i32) % (NP - N))
    row = jnp.concatenate([edge_index[0].astype(i32), loop, pad_row])
    col = jnp.concatenate([edge_index[1].astype(i32), loop, pad_col])
    row3 = row.reshape(NW, CPW, CHUNK)
    col3 = col.reshape(NW, CPW, CHUNK)

    ea0 = jnp.concatenate([edge_attr[:, 0].astype(i32),
                           jnp.full((N,), 4, i32),
                           jnp.zeros((npad,), i32)])
    ea1 = jnp.concatenate([edge_attr[:, 1].astype(i32),
                           jnp.zeros((N,), i32),
                           jnp.zeros((npad,), i32)])

    t1 = jnp.concatenate([p['ee1'] for p in params['layers']], axis=1)
    t2 = jnp.concatenate([p['ee2'] for p in params['layers']], axis=1)
    t1 = jnp.pad(t1, ((0, 0), (0, D - L))).astype(f32)
    t2 = jnp.pad(t2, ((0, 0), (0, D - L))).astype(f32)

    zeros128 = jnp.zeros((CHUNK, D), f32)
    batchp = jnp.pad(batch.astype(i32), (0, NP - N),
                     constant_values=G)[None, :]

    REP = 64
    etab = (t1[:, None, :] + t2[None, :, :]).reshape(15, D)
    etab = jnp.repeat(etab, REP, axis=0)
    a3 = ((ea0 * 3 + ea1) * REP
          + (jnp.arange(EP, dtype=i32) % REP)).reshape(NW, CPW, CHUNK)
    eesum = _get_sc_edge()(etab, a3, col3, zeros128)

    xw = pl.pallas_call(
        _tc_embed_body,
        out_shape=jax.ShapeDtypeStruct((NP, D), f32),
    )(x0p, x1p, params['emb1'].astype(f32), params['emb2'].astype(f32),
      params['layers'][0]['W'].astype(f32))

    for l in range(L):
        part = _get_sc_edge()(xw, row3, col3, zeros128)
        p = params['layers'][l]
        w_next = (params['layers'][l + 1]['W'] if l < L - 1
                  else params['layers'][0]['W']).astype(f32)
        xw = pl.pallas_call(
            functools.partial(_tc_bn_body, l),
            out_shape=jax.ShapeDtypeStruct((NP, D), f32),
        )(part, eesum, p['b'][None, :].astype(f32),
          p['gamma'][None, :].astype(f32), p['beta'][None, :].astype(f32),
          w_next)

    out = pl.pallas_call(
        _tc_pool_body,
        out_shape=jax.ShapeDtypeStruct((G, 1), f32),
    )(xw, batchp, params['feat_W'].astype(f32), params['feat_b'][None, :].astype(f32),
      params['h1_W'].astype(f32), params['h1_b'][None, :].astype(f32),
      params['h2_W'].astype(f32), params['h2_b'][None, :].astype(f32))
    return out
